# Initial kernel scaffold; baseline (speedup 1.0000x reference)
import jax
import jax.numpy as jnp
from jax.experimental import pallas as pl


def kernel(x, edge_index, batch, W_rel1, b_rel1, W_root1, bn1_g, bn1_b, bn1_rm, bn1_rv, W_rel2, b_rel2, W_root2, bn2_g, bn2_b, bn2_rm, bn2_rv, W_rel3, b_rel3, W_root3, bn3_g, bn3_b, bn3_rm, bn3_rv, Wc, bc):
    raise NotImplementedError("write your pallas kernel here")


# speedup vs baseline: 2.6535x; 2.6535x over previous
import functools

import jax
import jax.numpy as jnp
from jax import lax
from jax.experimental import pallas as pl
from jax.experimental.pallas import tpu as pltpu
from jax.experimental.pallas import tpu_sc as plsc

N = 10000
E = 320000
D = 128
H = 128
C = 10
G = 64

NC = 2
NS = 16
NW = NC * NS
CH = 128
EPW = 10240
EP = NW * EPW
NCHUNK = EPW // CH
NTAB = 10112
RPT = NTAB // NS


def _sc_segment_sum(table, src3, dst3):
    mesh = plsc.VectorSubcoreMesh(core_axis_name="c", subcore_axis_name="s")

    @functools.partial(
        pl.kernel,
        out_type=jax.ShapeDtypeStruct((NC * NTAB, H), jnp.float32),
        mesh=mesh,
        scratch_types=[
            pltpu.VMEM((NCHUNK, CH), jnp.int32),
            pltpu.VMEM((NCHUNK, CH), jnp.int32),
            pltpu.VMEM((CH, H), jnp.float32),
            pltpu.VMEM_SHARED((NTAB, H), jnp.float32),
            pltpu.SemaphoreType.DMA,
        ],
    )
    def k(table_hbm, src_hbm, dst_hbm, out_hbm, sidx, didx, rows, acc, sem):
        c = lax.axis_index("c")
        s = lax.axis_index("s")
        wid = s * NC + c

        pltpu.sync_copy(src_hbm.at[wid], sidx)
        pltpu.sync_copy(dst_hbm.at[wid], didx)

        @pl.loop(0, CH)
        def _(i):
            for j in range(0, H, 16):
                rows.at[i, pl.ds(j, 16)][...] = jnp.zeros((16,), jnp.float32)

        z0 = s * RPT

        @pl.loop(0, RPT // CH)
        def _(kk):
            pltpu.sync_copy(rows, acc.at[pl.ds(z0 + kk * CH, CH)])

        rem = RPT % CH
        if rem:
            pltpu.sync_copy(
                rows.at[pl.ds(0, rem)],
                acc.at[pl.ds(z0 + (RPT // CH) * CH, rem)],
            )

        plsc.subcore_barrier()

        @pl.loop(0, NCHUNK)
        def _(j):
            pltpu.async_copy(table_hbm.at[sidx.at[j]], rows, sem).wait()
            pltpu.sync_copy(rows, acc.at[didx.at[j]], add=True)

        plsc.subcore_barrier()

        pltpu.sync_copy(
            acc.at[pl.ds(z0, RPT)],
            out_hbm.at[pl.ds(c * NTAB + z0, RPT)],
        )

    return k(table, src3, dst3)


def _tc_layer(a0, a1, h, A, B, bias, relu):
    def body(a0_ref, a1_ref, h_ref, A_ref, B_ref, b_ref, o_ref):
        z = jnp.dot(a0_ref[...] + a1_ref[...], A_ref[...],
                    preferred_element_type=jnp.float32)
        z = z + jnp.dot(h_ref[...], B_ref[...],
                        preferred_element_type=jnp.float32)
        z = z + b_ref[...]
        if relu:
            z = jnp.maximum(z, 0.0)
        o_ref[...] = z

    return pl.pallas_call(
        body,
        out_shape=jax.ShapeDtypeStruct((N, H), jnp.float32),
    )(a0, a1, h, A, B, bias)


def _tc_final(a0, a1, h, A, B, bias, batch_b, WcT, bc):
    def body(a0_ref, a1_ref, h_ref, A_ref, B_ref, b_ref, bat_ref, Wc_ref,
             bc_ref, o_ref):
        z = jnp.dot(a0_ref[...] + a1_ref[...], A_ref[...],
                    preferred_element_type=jnp.float32)
        z = z + jnp.dot(h_ref[...], B_ref[...],
                        preferred_element_type=jnp.float32)
        z = z + b_ref[...]
        gid = lax.broadcasted_iota(jnp.int32, (G, N), 0)
        onehot = (bat_ref[...] == gid).astype(jnp.float32)
        sums = jnp.dot(onehot, z, preferred_element_type=jnp.float32)
        cnt = jnp.sum(onehot, axis=1)
        pooled = sums / jnp.maximum(cnt, 1.0)[:, None]
        o_ref[...] = jnp.dot(pooled, Wc_ref[...],
                             preferred_element_type=jnp.float32) + bc_ref[...]

    return pl.pallas_call(
        body,
        out_shape=jax.ShapeDtypeStruct((G, H), jnp.float32),
    )(a0, a1, h, A, B, bias, batch_b, WcT, bc)


def _fold_bn(W_rel, b_rel, W_root, g, b, rm, rv):
    s = g / jnp.sqrt(rv + 1e-5)
    A = W_rel.T * s[None, :]
    B = W_root.T * s[None, :]
    bias = (b_rel * s + b - rm * s)[None, :]
    return A, B, bias


def kernel(x, edge_index, batch,
           W_rel1, b_rel1, W_root1, bn1_g, bn1_b, bn1_rm, bn1_rv,
           W_rel2, b_rel2, W_root2, bn2_g, bn2_b, bn2_rm, bn2_rv,
           W_rel3, b_rel3, W_root3, bn3_g, bn3_b, bn3_rm, bn3_rv,
           Wc, bc):
    src = edge_index[0]
    dst = edge_index[1]
    pad = EP - E
    src_p = jnp.concatenate([src, jnp.zeros((pad,), jnp.int32)])
    dst_p = jnp.concatenate([dst, jnp.full((pad,), N, jnp.int32)])
    src3 = src_p.reshape(NW, NCHUNK, CH)
    dst3 = dst_p.reshape(NW, NCHUNK, CH)

    A1, B1, bias1 = _fold_bn(W_rel1, b_rel1, W_root1, bn1_g, bn1_b, bn1_rm, bn1_rv)
    A2, B2, bias2 = _fold_bn(W_rel2, b_rel2, W_root2, bn2_g, bn2_b, bn2_rm, bn2_rv)
    A3, B3, bias3 = _fold_bn(W_rel3, b_rel3, W_root3, bn3_g, bn3_b, bn3_rm, bn3_rv)

    WcT = jnp.zeros((H, H), jnp.float32).at[:, :C].set(Wc.T)
    bc_p = jnp.zeros((1, H), jnp.float32).at[0, :C].set(bc)
    batch_b = jnp.broadcast_to(batch[None, :], (G, N))

    parts = _sc_segment_sum(x, src3, dst3)
    h1 = _tc_layer(parts[:N], parts[NTAB:NTAB + N], x, A1, B1, bias1, True)
    parts = _sc_segment_sum(h1, src3, dst3)
    h2 = _tc_layer(parts[:N], parts[NTAB:NTAB + N], h1, A2, B2, bias2, True)
    parts = _sc_segment_sum(h2, src3, dst3)
    out = _tc_final(parts[:N], parts[NTAB:NTAB + N], h2, A3, B3, bias3,
                    batch_b, WcT, bc_p)
    return out[:, :C]


# speedup vs baseline: 2.9327x; 1.1052x over previous
import functools

import jax
import jax.numpy as jnp
from jax import lax
from jax.experimental import pallas as pl
from jax.experimental.pallas import tpu as pltpu
from jax.experimental.pallas import tpu_sc as plsc

N = 10000
E = 320000
D = 128
H = 128
C = 10
G = 64

NC = 2
NS = 16
NW = NC * NS
CH = 128
EPW = 10240
EP = NW * EPW
NCHUNK = EPW // CH
NHALF = NCHUNK // 2
NTAB = 10112
RPT = NTAB // NS


def _sc_segment_sum(table, src3, dst3):
    mesh = plsc.VectorSubcoreMesh(core_axis_name="c", subcore_axis_name="s")

    @functools.partial(
        pl.kernel,
        out_type=jax.ShapeDtypeStruct((NC * NTAB, H), jnp.float32),
        mesh=mesh,
        scratch_types=[
            pltpu.VMEM((NCHUNK, CH), jnp.int32),
            pltpu.VMEM((NHALF, CH), jnp.int32),
            pltpu.VMEM((CH, H), jnp.float32),
            pltpu.VMEM((CH, H), jnp.float32),
            pltpu.VMEM_SHARED((NTAB, H), jnp.float32),
            pltpu.SemaphoreType.DMA,
            pltpu.SemaphoreType.DMA,
        ],
    )
    def k(table_hbm, src_hbm, dst_hbm, out_hbm,
          sidx, didx, rows0, rows1, acc, sem0, sem1):
        c = lax.axis_index("c")
        s = lax.axis_index("s")
        wid = s * NC + c
        z0 = s * RPT

        pltpu.sync_copy(src_hbm.at[wid], sidx)

        @pl.loop(0, CH)
        def _(i):
            for j in range(0, H, 16):
                rows0.at[i, pl.ds(j, 16)][...] = jnp.zeros((16,), jnp.float32)

        @pl.loop(0, RPT // CH)
        def _(kk):
            pltpu.sync_copy(rows0, acc.at[pl.ds(z0 + kk * CH, CH)])

        rem = RPT % CH
        if rem:
            pltpu.sync_copy(
                rows0.at[pl.ds(0, rem)],
                acc.at[pl.ds(z0 + (RPT // CH) * CH, rem)],
            )

        plsc.subcore_barrier()

        for half in range(2):
            h0 = half * NHALF
            pltpu.sync_copy(dst_hbm.at[wid, pl.ds(h0, NHALF)], didx)
            pltpu.make_async_copy(
                table_hbm.at[sidx.at[h0]], rows0, sem0).start()

            @pl.loop(0, NHALF, step=2)
            def _(j):
                pltpu.make_async_copy(
                    table_hbm.at[sidx.at[h0 + j + 1]], rows1, sem1).start()
                pltpu.make_async_copy(
                    table_hbm.at[sidx.at[h0 + j]], rows0, sem0).wait()
                pltpu.sync_copy(rows0, acc.at[didx.at[j]], add=True)

                @pl.when(j + 2 < NHALF)
                def _():
                    pltpu.make_async_copy(
                        table_hbm.at[sidx.at[h0 + j + 2]], rows0, sem0).start()

                pltpu.make_async_copy(
                    table_hbm.at[sidx.at[h0 + j + 1]], rows1, sem1).wait()
                pltpu.sync_copy(rows1, acc.at[didx.at[j + 1]], add=True)

        plsc.subcore_barrier()

        pltpu.sync_copy(
            acc.at[pl.ds(z0, RPT)],
            out_hbm.at[pl.ds(c * NTAB + z0, RPT)],
        )

    return k(table, src3, dst3)


def _tc_layer(a0, a1, h, A, B, bias, relu):
    def body(a0_ref, a1_ref, h_ref, A_ref, B_ref, b_ref, o_ref):
        z = jnp.dot(a0_ref[...] + a1_ref[...], A_ref[...],
                    preferred_element_type=jnp.float32)
        z = z + jnp.dot(h_ref[...], B_ref[...],
                        preferred_element_type=jnp.float32)
        z = z + b_ref[...]
        if relu:
            z = jnp.maximum(z, 0.0)
        o_ref[...] = z

    return pl.pallas_call(
        body,
        out_shape=jax.ShapeDtypeStruct((N, H), jnp.float32),
    )(a0, a1, h, A, B, bias)


def _tc_final(a0, a1, h, A, B, bias, batch_b, WcT, bc):
    def body(a0_ref, a1_ref, h_ref, A_ref, B_ref, b_ref, bat_ref, Wc_ref,
             bc_ref, o_ref):
        z = jnp.dot(a0_ref[...] + a1_ref[...], A_ref[...],
                    preferred_element_type=jnp.float32)
        z = z + jnp.dot(h_ref[...], B_ref[...],
                        preferred_element_type=jnp.float32)
        z = z + b_ref[...]
        gid = lax.broadcasted_iota(jnp.int32, (G, N), 0)
        onehot = (bat_ref[...] == gid).astype(jnp.float32)
        sums = jnp.dot(onehot, z, preferred_element_type=jnp.float32)
        cnt = jnp.sum(onehot, axis=1)
        pooled = sums / jnp.maximum(cnt, 1.0)[:, None]
        o_ref[...] = jnp.dot(pooled, Wc_ref[...],
                             preferred_element_type=jnp.float32) + bc_ref[...]

    return pl.pallas_call(
        body,
        out_shape=jax.ShapeDtypeStruct((G, H), jnp.float32),
    )(a0, a1, h, A, B, bias, batch_b, WcT, bc)


def _fold_bn(W_rel, b_rel, W_root, g, b, rm, rv):
    s = g / jnp.sqrt(rv + 1e-5)
    A = W_rel.T * s[None, :]
    B = W_root.T * s[None, :]
    bias = (b_rel * s + b - rm * s)[None, :]
    return A, B, bias


def kernel(x, edge_index, batch,
           W_rel1, b_rel1, W_root1, bn1_g, bn1_b, bn1_rm, bn1_rv,
           W_rel2, b_rel2, W_root2, bn2_g, bn2_b, bn2_rm, bn2_rv,
           W_rel3, b_rel3, W_root3, bn3_g, bn3_b, bn3_rm, bn3_rv,
           Wc, bc):
    src = edge_index[0]
    dst = edge_index[1]
    pad = EP - E
    src_p = jnp.concatenate([src, jnp.zeros((pad,), jnp.int32)])
    dst_p = jnp.concatenate([dst, jnp.full((pad,), N, jnp.int32)])
    src3 = src_p.reshape(NW, NCHUNK, CH)
    dst3 = dst_p.reshape(NW, NCHUNK, CH)

    A1, B1, bias1 = _fold_bn(W_rel1, b_rel1, W_root1, bn1_g, bn1_b, bn1_rm, bn1_rv)
    A2, B2, bias2 = _fold_bn(W_rel2, b_rel2, W_root2, bn2_g, bn2_b, bn2_rm, bn2_rv)
    A3, B3, bias3 = _fold_bn(W_rel3, b_rel3, W_root3, bn3_g, bn3_b, bn3_rm, bn3_rv)

    WcT = jnp.zeros((H, H), jnp.float32).at[:, :C].set(Wc.T)
    bc_p = jnp.zeros((1, H), jnp.float32).at[0, :C].set(bc)
    batch_b = jnp.broadcast_to(batch[None, :], (G, N))

    parts = _sc_segment_sum(x, src3, dst3)
    h1 = _tc_layer(parts[:N], parts[NTAB:NTAB + N], x, A1, B1, bias1, True)
    parts = _sc_segment_sum(h1, src3, dst3)
    h2 = _tc_layer(parts[:N], parts[NTAB:NTAB + N], h1, A2, B2, bias2, True)
    parts = _sc_segment_sum(h2, src3, dst3)
    out = _tc_final(parts[:N], parts[NTAB:NTAB + N], h2, A3, B3, bias3,
                    batch_b, WcT, bc_p)
    return out[:, :C]


# speedup vs baseline: 3.0819x; 1.0509x over previous
import functools

import jax
import jax.numpy as jnp
from jax import lax
from jax.experimental import pallas as pl
from jax.experimental.pallas import tpu as pltpu
from jax.experimental.pallas import tpu_sc as plsc

N = 10000
E = 320000
D = 128
H = 128
C = 10
G = 64

NC = 2
NS = 16
CH = 128
QC = 32
QE = QC * CH
NQ0 = 4
NQ1 = 1
E0 = NS * NQ0 * QE
E1 = NS * NQ1 * QE
EP = E0 + E1
NTAB = 10112
RPT = NTAB // NS


def _sc_segment_sum(table, srcA, dstA, srcB, dstB):
    mesh = plsc.VectorSubcoreMesh(core_axis_name="c", subcore_axis_name="s")

    @functools.partial(
        pl.kernel,
        out_type=jax.ShapeDtypeStruct((NC * NTAB, H), jnp.float32),
        mesh=mesh,
        scratch_types=[
            pltpu.VMEM((QC, CH), jnp.int32),
            pltpu.VMEM((QC, CH), jnp.int32),
            pltpu.VMEM((CH, H), jnp.float32),
            pltpu.VMEM((CH, H), jnp.float32),
            pltpu.VMEM_SHARED((NTAB, H), jnp.float32),
            pltpu.SemaphoreType.DMA,
            pltpu.SemaphoreType.DMA,
        ],
    )
    def k(table_hbm, srcA_hbm, dstA_hbm, srcB_hbm, dstB_hbm, out_hbm,
          sidx, didx, rows0, rows1, acc, sem0, sem1):
        c = lax.axis_index("c")
        s = lax.axis_index("s")
        z0 = s * RPT

        @pl.loop(0, CH)
        def _(i):
            for j in range(0, H, 16):
                rows0.at[i, pl.ds(j, 16)][...] = jnp.zeros((16,), jnp.float32)

        @pl.loop(0, RPT // CH)
        def _(kk):
            pltpu.sync_copy(rows0, acc.at[pl.ds(z0 + kk * CH, CH)])

        rem = RPT % CH
        if rem:
            pltpu.sync_copy(
                rows0.at[pl.ds(0, rem)],
                acc.at[pl.ds(z0 + (RPT // CH) * CH, rem)],
            )

        plsc.subcore_barrier()

        def quarter(src_hbm, dst_hbm, q):
            pltpu.sync_copy(src_hbm.at[s, q], sidx)
            pltpu.sync_copy(dst_hbm.at[s, q], didx)
            pltpu.make_async_copy(
                table_hbm.at[sidx.at[0]], rows0, sem0).start()

            @pl.loop(0, QC, step=2)
            def _(j):
                pltpu.make_async_copy(
                    table_hbm.at[sidx.at[j + 1]], rows1, sem1).start()
                pltpu.make_async_copy(
                    table_hbm.at[sidx.at[j]], rows0, sem0).wait()
                pltpu.sync_copy(rows0, acc.at[didx.at[j]], add=True)

                @pl.when(j + 2 < QC)
                def _():
                    pltpu.make_async_copy(
                        table_hbm.at[sidx.at[j + 2]], rows0, sem0).start()

                pltpu.make_async_copy(
                    table_hbm.at[sidx.at[j + 1]], rows1, sem1).wait()
                pltpu.sync_copy(rows1, acc.at[didx.at[j + 1]], add=True)

        @pl.when(c == 0)
        def _():
            for q in range(NQ0):
                quarter(srcA_hbm, dstA_hbm, q)

        @pl.when(c == 1)
        def _():
            for q in range(NQ1):
                quarter(srcB_hbm, dstB_hbm, q)

        plsc.subcore_barrier()

        pltpu.sync_copy(
            acc.at[pl.ds(z0, RPT)],
            out_hbm.at[pl.ds(c * NTAB + z0, RPT)],
        )

    return k(table, srcA, dstA, srcB, dstB)


def _tc_layer(a0, a1, h, A, B, bias, relu):
    def body(a0_ref, a1_ref, h_ref, A_ref, B_ref, b_ref, o_ref):
        z = jnp.dot(a0_ref[...] + a1_ref[...], A_ref[...],
                    preferred_element_type=jnp.float32)
        z = z + jnp.dot(h_ref[...], B_ref[...],
                        preferred_element_type=jnp.float32)
        z = z + b_ref[...]
        if relu:
            z = jnp.maximum(z, 0.0)
        o_ref[...] = z

    return pl.pallas_call(
        body,
        out_shape=jax.ShapeDtypeStruct((N, H), jnp.float32),
    )(a0, a1, h, A, B, bias)


def _tc_final(a0, a1, h, A, B, bias, batch_b, WcT, bc):
    def body(a0_ref, a1_ref, h_ref, A_ref, B_ref, b_ref, bat_ref, Wc_ref,
             bc_ref, o_ref):
        z = jnp.dot(a0_ref[...] + a1_ref[...], A_ref[...],
                    preferred_element_type=jnp.float32)
        z = z + jnp.dot(h_ref[...], B_ref[...],
                        preferred_element_type=jnp.float32)
        z = z + b_ref[...]
        gid = lax.broadcasted_iota(jnp.int32, (G, N), 0)
        onehot = (bat_ref[...] == gid).astype(jnp.float32)
        sums = jnp.dot(onehot, z, preferred_element_type=jnp.float32)
        cnt = jnp.sum(onehot, axis=1)
        pooled = sums / jnp.maximum(cnt, 1.0)[:, None]
        o_ref[...] = jnp.dot(pooled, Wc_ref[...],
                             preferred_element_type=jnp.float32) + bc_ref[...]

    return pl.pallas_call(
        body,
        out_shape=jax.ShapeDtypeStruct((G, H), jnp.float32),
    )(a0, a1, h, A, B, bias, batch_b, WcT, bc)


def _fold_bn(W_rel, b_rel, W_root, g, b, rm, rv):
    s = g / jnp.sqrt(rv + 1e-5)
    A = W_rel.T * s[None, :]
    B = W_root.T * s[None, :]
    bias = (b_rel * s + b - rm * s)[None, :]
    return A, B, bias


def kernel(x, edge_index, batch,
           W_rel1, b_rel1, W_root1, bn1_g, bn1_b, bn1_rm, bn1_rv,
           W_rel2, b_rel2, W_root2, bn2_g, bn2_b, bn2_rm, bn2_rv,
           W_rel3, b_rel3, W_root3, bn3_g, bn3_b, bn3_rm, bn3_rv,
           Wc, bc):
    src = edge_index[0]
    dst = edge_index[1]
    pad = EP - E
    src_p = jnp.concatenate([src, jnp.zeros((pad,), jnp.int32)])
    dst_p = jnp.concatenate([dst, jnp.full((pad,), N, jnp.int32)])
    srcB = src_p[:E1].reshape(NS, NQ1, QC, CH)
    dstB = dst_p[:E1].reshape(NS, NQ1, QC, CH)
    srcA = src_p[E1:].reshape(NS, NQ0, QC, CH)
    dstA = dst_p[E1:].reshape(NS, NQ0, QC, CH)

    A1, B1, bias1 = _fold_bn(W_rel1, b_rel1, W_root1, bn1_g, bn1_b, bn1_rm, bn1_rv)
    A2, B2, bias2 = _fold_bn(W_rel2, b_rel2, W_root2, bn2_g, bn2_b, bn2_rm, bn2_rv)
    A3, B3, bias3 = _fold_bn(W_rel3, b_rel3, W_root3, bn3_g, bn3_b, bn3_rm, bn3_rv)

    WcT = jnp.zeros((H, H), jnp.float32).at[:, :C].set(Wc.T)
    bc_p = jnp.zeros((1, H), jnp.float32).at[0, :C].set(bc)
    batch_b = jnp.broadcast_to(batch[None, :], (G, N))

    parts = _sc_segment_sum(x, srcA, dstA, srcB, dstB)
    h1 = _tc_layer(parts[:N], parts[NTAB:NTAB + N], x, A1, B1, bias1, True)
    parts = _sc_segment_sum(h1, srcA, dstA, srcB, dstB)
    h2 = _tc_layer(parts[:N], parts[NTAB:NTAB + N], h1, A2, B2, bias2, True)
    parts = _sc_segment_sum(h2, srcA, dstA, srcB, dstB)
    out = _tc_final(parts[:N], parts[NTAB:NTAB + N], h2, A3, B3, bias3,
                    batch_b, WcT, bc_p)
    return out[:, :C]
